# baseline (speedup 1.0000x reference)
import functools

import jax
import jax.numpy as jnp
import numpy as np
from jax import lax
from jax.experimental import pallas as pl
from jax.experimental.pallas import tpu as pltpu
from jax.experimental.pallas import tpu_sc as plsc

H = 768
HV = H // 16
NW = 32
C = 32


_DNUMS = lax.GatherDimensionNumbers(
    offset_dims=(), collapsed_slice_dims=(0,), start_index_map=(0,))
def _allsum_vec(v):
    lanes = lax.broadcasted_iota(jnp.int32, (16,), 0)
    for m in (8, 4, 2, 1):
        perm = lax.reshape(lanes ^ m, (16, 1))
        v = v + lax.gather(v, perm, _DNUMS, slice_sizes=(1,),
                           mode=lax.GatherScatterMode.PROMISE_IN_BOUNDS)
    return v


def _rsqrt_vec(x):
    xs = x[0]
    i = lax.bitcast_convert_type(xs, jnp.int32)
    i = jnp.int32(0x5F3759DF) - lax.shift_right_logical(i, 1)
    ys = lax.bitcast_convert_type(i, jnp.float32)
    y = jnp.full((16,), ys, jnp.float32)
    for _ in range(3):
        y = y * (1.5 - 0.5 * x * y * y)
    return y


def _sc_body(ids_hbm, tids_hbm, word_hbm, base_hbm, gam_hbm, bet_hbm,
             out_hbm,
             ids_v, tids_v, base_v, gam_v, bet_v,
             in0, in1, out0, out1,
             gsem0, gsem1, ssem0, ssem1):
    nc = 2
    wid = lax.axis_index("s") * nc + lax.axis_index("c")
    n_tok = ids_hbm.shape[0]
    tpw = n_tok // NW
    nch = tpw // C
    tok0 = wid * tpw

    pltpu.sync_copy(ids_hbm.at[pl.ds(tok0, tpw)], ids_v)
    pltpu.sync_copy(tids_hbm.at[pl.ds(tok0, tpw)], tids_v)
    pltpu.sync_copy(base_hbm, base_v)
    pltpu.sync_copy(gam_hbm, gam_v)
    pltpu.sync_copy(bet_hbm, bet_v)

    ins = (in0, in1)
    outs = (out0, out1)
    gsems = (gsem0, gsem1)
    ssems = (ssem0, ssem1)

    def gather_desc(g, s):
        idx = ids_v.at[pl.ds(g * C, C)]
        return pltpu.make_async_copy(word_hbm.at[idx], ins[s], gsems[s])

    def scatter_desc(g, s):
        return pltpu.make_async_copy(
            outs[s], out_hbm.at[pl.ds(tok0 + g * C, C)], ssems[s])

    def compute(g, s):
        in_ref = ins[s]
        out_ref = outs[s]

        def row_body(r, carry):
            tvec = tids_v[pl.ds(g * C + (r & ~15), 16)]
            lane_idx = lax.reshape(jnp.full((16,), r & 15, jnp.int32),
                                   (16, 1))
            tidv = lax.gather(tvec, lane_idx, _DNUMS, slice_sizes=(1,),
                              mode=lax.GatherScatterMode.PROMISE_IN_BOUNDS)
            tf = tidv.astype(jnp.float32)
            acc1 = jnp.zeros((16,), jnp.float32)
            acc2 = jnp.zeros((16,), jnp.float32)
            for k in range(HV):
                w = in_ref[r, pl.ds(k * 16, 16)]
                b0 = base_v[pl.ds(k * 16, 16)]
                b1 = base_v[pl.ds(H + k * 16, 16)]
                x = w + (b0 + tf * (b1 - b0))
                acc1 = acc1 + x
                acc2 = acc2 + x * x
            mean = _allsum_vec(acc1) * (1.0 / H)
            var = _allsum_vec(acc2) * (1.0 / H) - mean * mean + 1e-5
            inv = _rsqrt_vec(var)
            for k in range(HV):
                w = in_ref[r, pl.ds(k * 16, 16)]
                b0 = base_v[pl.ds(k * 16, 16)]
                b1 = base_v[pl.ds(H + k * 16, 16)]
                x = (w + (b0 + tf * (b1 - b0)) - mean) * inv
                y = x * gam_v[pl.ds(k * 16, 16)] + bet_v[pl.ds(k * 16, 16)]
                out_ref[r, pl.ds(k * 16, 16)] = y
            return carry

        lax.fori_loop(0, C, row_body, 0)

    gather_desc(0, 0).start()
    gather_desc(1, 1).start()

    def chunk_iter(i, carry):
        for s in range(2):
            g = i * 2 + s
            gather_desc(g, s).wait()

            @pl.when(g >= 2)
            def _():
                scatter_desc(g - 2, s).wait()

            compute(g, s)
            scatter_desc(g, s).start()

            @pl.when(g + 2 < nch)
            def _():
                gather_desc(g + 2, s).start()
        return carry

    lax.fori_loop(0, nch // 2, chunk_iter, 0)
    scatter_desc(nch - 2, 0).wait()
    scatter_desc(nch - 1, 1).wait()


@jax.jit
def _fwd(ids, tids, word_emb, base, gamma, beta):
    n = ids.shape[0]
    tpw = n // NW
    mesh = plsc.VectorSubcoreMesh(core_axis_name="c", subcore_axis_name="s")
    run = pl.kernel(
        _sc_body,
        out_type=jax.ShapeDtypeStruct((n, H), jnp.float32),
        mesh=mesh,
        scratch_types=[
            pltpu.VMEM((tpw,), jnp.int32),
            pltpu.VMEM((tpw,), jnp.int32),
            pltpu.VMEM((2 * H,), jnp.float32),
            pltpu.VMEM((H,), jnp.float32),
            pltpu.VMEM((H,), jnp.float32),
            pltpu.VMEM((C, H), jnp.float32),
            pltpu.VMEM((C, H), jnp.float32),
            pltpu.VMEM((C, H), jnp.float32),
            pltpu.VMEM((C, H), jnp.float32),
            pltpu.SemaphoreType.DMA,
            pltpu.SemaphoreType.DMA,
            pltpu.SemaphoreType.DMA,
            pltpu.SemaphoreType.DMA,
        ],
    )
    return run(ids, tids, word_emb, base, gamma, beta)


def kernel(input_ids, token_type_ids, word_emb, type_emb, pos_emb,
           ln_gamma, ln_beta):
    b, s1, s = input_ids.shape
    ids = input_ids.reshape(-1).astype(jnp.int32)
    tids = token_type_ids.reshape(-1).astype(jnp.int32)
    base = (type_emb + pos_emb[0][None, :]).reshape(-1)
    out = _fwd(ids, tids, word_emb, base,
               ln_gamma.astype(jnp.float32), ln_beta.astype(jnp.float32))
    return out.reshape(b, s1, s, H)


# speedup vs baseline: 4.1850x; 4.1850x over previous
import functools

import jax
import jax.numpy as jnp
import numpy as np
from jax import lax
from jax.experimental import pallas as pl
from jax.experimental.pallas import tpu as pltpu
from jax.experimental.pallas import tpu_sc as plsc

H = 768
NW = 32
C = 64
RB = 512


def _gather_body(ids_hbm, word_hbm, out_hbm, ids_v, buf0, buf1,
                 gsem0, gsem1, ssem0, ssem1):
    nc = 2
    wid = lax.axis_index("s") * nc + lax.axis_index("c")
    n_tok = ids_hbm.shape[0]
    tpw = n_tok // NW
    nch = tpw // C
    tok0 = wid * tpw

    pltpu.sync_copy(ids_hbm.at[pl.ds(tok0, tpw)], ids_v)

    bufs = (buf0, buf1)
    gsems = (gsem0, gsem1)
    ssems = (ssem0, ssem1)

    def gather_desc(g, s):
        idx = ids_v.at[pl.ds(g * C, C)]
        return pltpu.make_async_copy(word_hbm.at[idx], bufs[s], gsems[s])

    def scatter_desc(g, s):
        return pltpu.make_async_copy(
            bufs[s], out_hbm.at[pl.ds(tok0 + g * C, C)], ssems[s])

    gather_desc(0, 0).start()
    gather_desc(1, 1).start()

    def round_iter(i, carry):
        g = i * 2
        for s in range(2):
            gather_desc(g + s, s).wait()
            scatter_desc(g + s, s).start()

        @pl.when(g + 2 < nch)
        def _():
            for s in range(2):
                scatter_desc(g + s, s).wait()
                gather_desc(g + 2 + s, s).start()
        return carry

    lax.fori_loop(0, nch // 2, round_iter, 0)
    scatter_desc(nch - 2, 0).wait()
    scatter_desc(nch - 1, 1).wait()


@jax.jit
def _sc_gather(ids, word_emb):
    n = ids.shape[0]
    tpw = n // NW
    mesh = plsc.VectorSubcoreMesh(core_axis_name="c", subcore_axis_name="s")
    run = pl.kernel(
        _gather_body,
        out_type=jax.ShapeDtypeStruct((n, H), jnp.float32),
        mesh=mesh,
        scratch_types=[
            pltpu.VMEM((tpw,), jnp.int32),
            pltpu.VMEM((C, H), jnp.float32),
            pltpu.VMEM((C, H), jnp.float32),
            pltpu.SemaphoreType.DMA,
            pltpu.SemaphoreType.DMA,
            pltpu.SemaphoreType.DMA,
            pltpu.SemaphoreType.DMA,
        ],
    )
    return run(ids, word_emb)


def _ln_body(x_ref, t_ref, b0_ref, bd_ref, gam_ref, bet_ref, o_ref):
    x = x_ref[...]
    t = t_ref[...]
    base = b0_ref[...] + t * bd_ref[...]
    x = x + base
    mean = jnp.mean(x, axis=-1, keepdims=True)
    xc = x - mean
    var = jnp.mean(xc * xc, axis=-1, keepdims=True)
    inv = lax.rsqrt(var + 1e-5)
    o_ref[...] = (xc * inv) * gam_ref[...] + bet_ref[...]


@jax.jit
def _tc_ln(rows, tf, b0, bd, gamma, beta):
    n = rows.shape[0]
    grid = (n // RB,)
    return pl.pallas_call(
        _ln_body,
        grid=grid,
        in_specs=[
            pl.BlockSpec((RB, H), lambda i: (i, 0)),
            pl.BlockSpec((RB, 1), lambda i: (i, 0)),
            pl.BlockSpec((1, H), lambda i: (0, 0)),
            pl.BlockSpec((1, H), lambda i: (0, 0)),
            pl.BlockSpec((1, H), lambda i: (0, 0)),
            pl.BlockSpec((1, H), lambda i: (0, 0)),
        ],
        out_specs=pl.BlockSpec((RB, H), lambda i: (i, 0)),
        out_shape=jax.ShapeDtypeStruct((n, H), jnp.float32),
    )(rows, tf, b0, bd, gamma, beta)


def kernel(input_ids, token_type_ids, word_emb, type_emb, pos_emb,
           ln_gamma, ln_beta):
    b, s1, s = input_ids.shape
    ids = input_ids.reshape(-1).astype(jnp.int32)
    tf = token_type_ids.reshape(-1, 1).astype(jnp.float32)
    b0 = (type_emb[0] + pos_emb[0]).reshape(1, H)
    bd = (type_emb[1] - type_emb[0]).reshape(1, H)
    rows = _sc_gather(ids, word_emb)
    out = _tc_ln(rows, tf, b0, bd,
                 ln_gamma.astype(jnp.float32).reshape(1, H),
                 ln_beta.astype(jnp.float32).reshape(1, H))
    return out.reshape(b, s1, s, H)


# speedup vs baseline: 4.4191x; 1.0560x over previous
import functools

import jax
import jax.numpy as jnp
import numpy as np
from jax import lax
from jax.experimental import pallas as pl
from jax.experimental.pallas import tpu as pltpu
from jax.experimental.pallas import tpu_sc as plsc

H = 768
NW = 32
C = 64
RB = 512


def _gather_body(ids_hbm, word_hbm, out_hbm, ids_v, buf0, buf1,
                 gsem0, gsem1, ssem0, ssem1):
    nc = 2
    wid = lax.axis_index("s") * nc + lax.axis_index("c")
    n_tok = ids_hbm.shape[0]
    tpw = n_tok // NW
    nch = tpw // C
    tok0 = wid * tpw

    pltpu.sync_copy(ids_hbm.at[pl.ds(tok0, tpw)], ids_v)

    bufs = (buf0, buf1)
    gsems = (gsem0, gsem1)
    ssems = (ssem0, ssem1)

    def gather_desc(g, s):
        idx = ids_v.at[pl.ds(g * C, C)]
        return pltpu.make_async_copy(word_hbm.at[idx], bufs[s], gsems[s])

    def scatter_desc(g, s):
        return pltpu.make_async_copy(
            bufs[s], out_hbm.at[pl.ds(tok0 + g * C, C)], ssems[s])

    gather_desc(0, 0).start()
    gather_desc(1, 1).start()

    def round_iter(i, carry):
        g = i * 2
        for s in range(2):
            gather_desc(g + s, s).wait()
            scatter_desc(g + s, s).start()

        @pl.when(g + 2 < nch)
        def _():
            for s in range(2):
                scatter_desc(g + s, s).wait()
                gather_desc(g + 2 + s, s).start()
        return carry

    lax.fori_loop(0, nch // 2, round_iter, 0)
    scatter_desc(nch - 2, 0).wait()
    scatter_desc(nch - 1, 1).wait()


@jax.jit
def _sc_gather(ids, word_emb):
    n = ids.shape[0]
    tpw = n // NW
    mesh = plsc.VectorSubcoreMesh(core_axis_name="c", subcore_axis_name="s")
    run = pl.kernel(
        _gather_body,
        out_type=jax.ShapeDtypeStruct((n, H), jnp.float32),
        mesh=mesh,
        scratch_types=[
            pltpu.VMEM((tpw,), jnp.int32),
            pltpu.VMEM((C, H), jnp.float32),
            pltpu.VMEM((C, H), jnp.float32),
            pltpu.SemaphoreType.DMA,
            pltpu.SemaphoreType.DMA,
            pltpu.SemaphoreType.DMA,
            pltpu.SemaphoreType.DMA,
        ],
    )
    return run(ids, word_emb)


def _ln_body(x_ref, t_ref, b0_ref, bd_ref, gam_ref, bet_ref, o_ref):
    x = x_ref[...]
    t = t_ref[...]
    base = b0_ref[...] + t * bd_ref[...]
    x = x + base
    mean = jnp.mean(x, axis=-1, keepdims=True)
    xc = x - mean
    var = jnp.mean(xc * xc, axis=-1, keepdims=True)
    inv = lax.rsqrt(var + 1e-5)
    o_ref[...] = (xc * inv) * gam_ref[...] + bet_ref[...]


def _ln_body_acc(acc_ref, x_ref, t_ref, b0_ref, bd_ref, gam_ref, bet_ref,
                 o_ref):
    del acc_ref
    _ln_body(x_ref, t_ref, b0_ref, bd_ref, gam_ref, bet_ref, o_ref)


def _tc_ln_chunk(acc, rows_k, tf_k, b0, bd, gamma, beta, n_total, blk0):
    m = rows_k.shape[0]
    grid = (m // RB,)
    small = [
        pl.BlockSpec((RB, 1), lambda i: (i, 0)),
        pl.BlockSpec((1, H), lambda i: (0, 0)),
        pl.BlockSpec((1, H), lambda i: (0, 0)),
        pl.BlockSpec((1, H), lambda i: (0, 0)),
        pl.BlockSpec((1, H), lambda i: (0, 0)),
    ]
    rows_spec = pl.BlockSpec((RB, H), lambda i: (i, 0))
    out_spec = pl.BlockSpec((RB, H), lambda i: (i + blk0, 0))
    out_shape = jax.ShapeDtypeStruct((n_total, H), jnp.float32)
    if acc is None:
        return pl.pallas_call(
            _ln_body, grid=grid,
            in_specs=[rows_spec] + small,
            out_specs=out_spec, out_shape=out_shape,
        )(rows_k, tf_k, b0, bd, gamma, beta)
    return pl.pallas_call(
        _ln_body_acc, grid=grid,
        in_specs=[pl.BlockSpec(memory_space=pl.ANY), rows_spec] + small,
        out_specs=out_spec, out_shape=out_shape,
        input_output_aliases={0: 0},
    )(acc, rows_k, tf_k, b0, bd, gamma, beta)


NCHUNK = 4


def kernel(input_ids, token_type_ids, word_emb, type_emb, pos_emb,
           ln_gamma, ln_beta):
    b, s1, s = input_ids.shape
    n = b * s1 * s
    ids = input_ids.reshape(-1).astype(jnp.int32)
    tf = token_type_ids.reshape(-1, 1).astype(jnp.float32)
    b0 = (type_emb[0] + pos_emb[0]).reshape(1, H)
    bd = (type_emb[1] - type_emb[0]).reshape(1, H)
    gam = ln_gamma.astype(jnp.float32).reshape(1, H)
    bet = ln_beta.astype(jnp.float32).reshape(1, H)

    ck = n // NCHUNK
    rows = [_sc_gather(ids[k * ck:(k + 1) * ck], word_emb)
            for k in range(NCHUNK)]
    acc = None
    for k in range(NCHUNK):
        acc = _tc_ln_chunk(acc, rows[k], tf[k * ck:(k + 1) * ck],
                           b0, bd, gam, bet, n, k * (ck // RB))
    return acc.reshape(b, s1, s, H)


# speedup vs baseline: 4.5658x; 1.0332x over previous
import functools

import jax
import jax.numpy as jnp
import numpy as np
from jax import lax
from jax.experimental import pallas as pl
from jax.experimental.pallas import tpu as pltpu
from jax.experimental.pallas import tpu_sc as plsc

H = 768
NW = 32
C = 64
RB = 1024


def _gather_body(ids_hbm, word_hbm, out_hbm, ids_v, buf0, buf1,
                 gsem0, gsem1, ssem0, ssem1):
    nc = 2
    wid = lax.axis_index("s") * nc + lax.axis_index("c")
    n_tok = ids_hbm.shape[0]
    tpw = n_tok // NW
    nch = tpw // C
    tok0 = wid * tpw

    pltpu.sync_copy(ids_hbm.at[pl.ds(tok0, tpw)], ids_v)

    bufs = (buf0, buf1)
    gsems = (gsem0, gsem1)
    ssems = (ssem0, ssem1)

    def gather_desc(g, s):
        idx = ids_v.at[pl.ds(g * C, C)]
        return pltpu.make_async_copy(word_hbm.at[idx], bufs[s], gsems[s])

    def scatter_desc(g, s):
        return pltpu.make_async_copy(
            bufs[s], out_hbm.at[pl.ds(tok0 + g * C, C)], ssems[s])

    gather_desc(0, 0).start()
    gather_desc(1, 1).start()

    def round_iter(i, carry):
        g = i * 2
        for s in range(2):
            gather_desc(g + s, s).wait()
            scatter_desc(g + s, s).start()

        @pl.when(g + 2 < nch)
        def _():
            for s in range(2):
                scatter_desc(g + s, s).wait()
                gather_desc(g + 2 + s, s).start()
        return carry

    lax.fori_loop(0, nch // 2, round_iter, 0)
    scatter_desc(nch - 2, 0).wait()
    scatter_desc(nch - 1, 1).wait()


@jax.jit
def _sc_gather(ids, word_emb):
    n = ids.shape[0]
    tpw = n // NW
    mesh = plsc.VectorSubcoreMesh(core_axis_name="c", subcore_axis_name="s")
    run = pl.kernel(
        _gather_body,
        out_type=jax.ShapeDtypeStruct((n, H), jnp.float32),
        mesh=mesh,
        scratch_types=[
            pltpu.VMEM((tpw,), jnp.int32),
            pltpu.VMEM((C, H), jnp.float32),
            pltpu.VMEM((C, H), jnp.float32),
            pltpu.SemaphoreType.DMA,
            pltpu.SemaphoreType.DMA,
            pltpu.SemaphoreType.DMA,
            pltpu.SemaphoreType.DMA,
        ],
    )
    return run(ids, word_emb)


def _ln_body(x_ref, t_ref, b0_ref, bd_ref, gam_ref, bet_ref, o_ref):
    x = x_ref[...]
    t = t_ref[...]
    base = b0_ref[...] + t * bd_ref[...]
    x = x + base
    mean = jnp.mean(x, axis=-1, keepdims=True)
    xc = x - mean
    var = jnp.mean(xc * xc, axis=-1, keepdims=True)
    inv = lax.rsqrt(var + 1e-5)
    o_ref[...] = (xc * inv) * gam_ref[...] + bet_ref[...]


def _ln_body_acc(acc_ref, x_ref, t_ref, b0_ref, bd_ref, gam_ref, bet_ref,
                 o_ref):
    del acc_ref
    _ln_body(x_ref, t_ref, b0_ref, bd_ref, gam_ref, bet_ref, o_ref)


def _tc_ln_chunk(acc, rows_k, tf_k, b0, bd, gamma, beta, n_total, blk0):
    m = rows_k.shape[0]
    grid = (m // RB,)
    small = [
        pl.BlockSpec((RB, 1), lambda i: (i, 0)),
        pl.BlockSpec((1, H), lambda i: (0, 0)),
        pl.BlockSpec((1, H), lambda i: (0, 0)),
        pl.BlockSpec((1, H), lambda i: (0, 0)),
        pl.BlockSpec((1, H), lambda i: (0, 0)),
    ]
    rows_spec = pl.BlockSpec((RB, H), lambda i: (i, 0))
    out_spec = pl.BlockSpec((RB, H), lambda i: (i + blk0, 0))
    out_shape = jax.ShapeDtypeStruct((n_total, H), jnp.float32)
    if acc is None:
        return pl.pallas_call(
            _ln_body, grid=grid,
            in_specs=[rows_spec] + small,
            out_specs=out_spec, out_shape=out_shape,
        )(rows_k, tf_k, b0, bd, gamma, beta)
    return pl.pallas_call(
        _ln_body_acc, grid=grid,
        in_specs=[pl.BlockSpec(memory_space=pl.ANY), rows_spec] + small,
        out_specs=out_spec, out_shape=out_shape,
        input_output_aliases={0: 0},
    )(acc, rows_k, tf_k, b0, bd, gamma, beta)


NCHUNK = 2


def kernel(input_ids, token_type_ids, word_emb, type_emb, pos_emb,
           ln_gamma, ln_beta):
    b, s1, s = input_ids.shape
    n = b * s1 * s
    ids = input_ids.reshape(-1).astype(jnp.int32)
    tf = token_type_ids.reshape(-1, 1).astype(jnp.float32)
    b0 = (type_emb[0] + pos_emb[0]).reshape(1, H)
    bd = (type_emb[1] - type_emb[0]).reshape(1, H)
    gam = ln_gamma.astype(jnp.float32).reshape(1, H)
    bet = ln_beta.astype(jnp.float32).reshape(1, H)

    ck = n // NCHUNK
    rows = [_sc_gather(ids[k * ck:(k + 1) * ck], word_emb)
            for k in range(NCHUNK)]
    acc = None
    for k in range(NCHUNK):
        acc = _tc_ln_chunk(acc, rows[k], tf[k * ck:(k + 1) * ck],
                           b0, bd, gam, bet, n, k * (ck // RB))
    return acc.reshape(b, s1, s, H)
